# baseline (speedup 1.0000x reference)
import jax
import jax.numpy as jnp
from jax import lax
from jax.experimental import pallas as pl
from jax.experimental.pallas import tpu as pltpu

B = 256
OBS = 1024
H = 2048
DEPTH = 100
NOUT = 64
HEADW = 128

_CD = 2
_NCH = DEPTH // _CD
_CH = H // 4


def _mega_body(idx_smem, x_ref, w1_ref, b1_ref, w2_ref, b2_ref, wh_ref,
               bh_ref, idx2d_ref, stack_any, ns_any, logits_ref, value_ref,
               nidx_ref, top, h, p, buf, sem_g, sem_i0, sem_i1, sem_o0,
               sem_o1):
    sem_in = (sem_i0, sem_i1)
    sem_out = (sem_o0, sem_o1)

    def in_cp(c):
        return pltpu.make_async_copy(
            stack_any.at[pl.ds(c * _CD, _CD)], buf.at[c & 1], sem_in[c & 1])

    def out_cp(c):
        return pltpu.make_async_copy(
            buf.at[c & 1], ns_any.at[pl.ds(c * _CD, _CD)], sem_out[c & 1])

    in_cp(0).start()
    in_cp(1).start()

    def g_start(r, carry):
        d = idx_smem[r]
        pltpu.make_async_copy(stack_any.at[d, r], top.at[r], sem_g).start()
        return carry

    lax.fori_loop(0, B, g_start, 0)

    def g_drain(r, carry):
        d = idx_smem[r]
        pltpu.make_async_copy(stack_any.at[d, r], top.at[r], sem_g).wait()
        return carry

    lax.fori_loop(0, B, g_drain, 0)

    def mlp_piece(c):
        if c < 4:
            c0 = c * _CH
            h[:, c0:c0 + _CH] = jnp.tanh(
                jnp.dot(x_ref[...], w1_ref[:OBS, c0:c0 + _CH],
                        preferred_element_type=jnp.float32)
                + jnp.dot(top[...], w1_ref[OBS:, c0:c0 + _CH],
                          preferred_element_type=jnp.float32)
                + b1_ref[:, c0:c0 + _CH]
            )
        elif c < 8:
            c0 = (c - 4) * _CH
            p[:, c0:c0 + _CH] = jnp.tanh(
                jnp.dot(h[...], w2_ref[:, c0:c0 + _CH],
                        preferred_element_type=jnp.float32)
                + b2_ref[:, c0:c0 + _CH]
            )
        elif c == 8:
            ph = jnp.dot(p[...], wh_ref[...],
                         preferred_element_type=jnp.float32) + bh_ref[...]
            logits_ref[...] = ph[:, 3:3 + NOUT]
            value_ref[...] = ph[:, 3 + NOUT:4 + NOUT]
            s0 = ph[:, 0:1]
            s1 = ph[:, 1:2]
            s2 = ph[:, 2:3]
            op = jnp.where(s1 > s0, 1, 0)
            best = jnp.maximum(s0, s1)
            op = jnp.where(s2 > best, 2, op)
            nidx_ref[...] = jnp.maximum(idx2d_ref[...] + op - 1, 0)

    for c in range(_NCH):
        in_cp(c).wait()
        out_cp(c).start()
        if c + 2 < _NCH:
            out_cp(c).wait()
            in_cp(c + 2).start()
        mlp_piece(c)
    out_cp(_NCH - 2).wait()
    out_cp(_NCH - 1).wait()

    def s_start(r, carry):
        t = idx_smem[r] + 1
        pltpu.make_async_copy(p.at[r], ns_any.at[t, r], sem_g).start()
        return carry

    lax.fori_loop(0, B, s_start, 0)

    def s_drain(r, carry):
        t = idx_smem[r] + 1
        pltpu.make_async_copy(p.at[r], ns_any.at[t, r], sem_g).wait()
        return carry

    lax.fori_loop(0, B, s_drain, 0)


def _tc_mega(stack_t, stack_idx, x, w1, b1, w2, b2, wh, bh, idx2d):
    return pl.pallas_call(
        _mega_body,
        out_shape=(
            jax.ShapeDtypeStruct((DEPTH, B, H), jnp.float32),
            jax.ShapeDtypeStruct((B, NOUT), jnp.float32),
            jax.ShapeDtypeStruct((B, 1), jnp.float32),
            jax.ShapeDtypeStruct((B, 1), jnp.int32),
        ),
        in_specs=[
            pl.BlockSpec(memory_space=pltpu.SMEM),
            pl.BlockSpec(memory_space=pltpu.VMEM),
            pl.BlockSpec(memory_space=pltpu.VMEM),
            pl.BlockSpec(memory_space=pltpu.VMEM),
            pl.BlockSpec(memory_space=pltpu.VMEM),
            pl.BlockSpec(memory_space=pltpu.VMEM),
            pl.BlockSpec(memory_space=pltpu.VMEM),
            pl.BlockSpec(memory_space=pltpu.VMEM),
            pl.BlockSpec(memory_space=pltpu.VMEM),
            pl.BlockSpec(memory_space=pl.ANY),
        ],
        out_specs=[
            pl.BlockSpec(memory_space=pl.ANY),
            pl.BlockSpec(memory_space=pltpu.VMEM),
            pl.BlockSpec(memory_space=pltpu.VMEM),
            pl.BlockSpec(memory_space=pltpu.VMEM),
        ],
        scratch_shapes=[
            pltpu.VMEM((B, H), jnp.float32),
            pltpu.VMEM((B, H), jnp.float32),
            pltpu.VMEM((B, H), jnp.float32),
            pltpu.VMEM((2, _CD, B, H), jnp.float32),
            pltpu.SemaphoreType.DMA,
            pltpu.SemaphoreType.DMA,
            pltpu.SemaphoreType.DMA,
            pltpu.SemaphoreType.DMA,
            pltpu.SemaphoreType.DMA,
        ],
    )(stack_idx, x, w1, b1, w2, b2, wh, bh, idx2d, stack_t)


def kernel(x, stack, stack_idx, W1, b1, W2, b2, Ws, bs, Wp, bp, Wv, bv):
    idx2d = stack_idx.reshape(B, 1)
    stack_t = jnp.swapaxes(stack, 0, 1)

    wh = jnp.concatenate(
        [Ws, Wp, Wv, jnp.zeros((H, HEADW - NOUT - 4), jnp.float32)], axis=1)
    bh = jnp.concatenate(
        [bs, bp, bv, jnp.zeros((HEADW - NOUT - 4,), jnp.float32)]
    ).reshape(1, HEADW)

    new_stack_t, logits, value, nidx = _tc_mega(
        stack_t, stack_idx, x, W1, b1.reshape(1, H), W2, b2.reshape(1, H),
        wh, bh, idx2d,
    )
    new_stack = jnp.swapaxes(new_stack_t, 0, 1)

    return (logits, value[:, 0], new_stack, nidx[:, 0])


# speedup vs baseline: 1.0768x; 1.0768x over previous
import jax
import jax.numpy as jnp
from jax import lax
from jax.experimental import pallas as pl
from jax.experimental.pallas import tpu as pltpu

B = 256
OBS = 1024
H = 2048
DEPTH = 100
NOUT = 64
HEADW = 128

_CD = 1
_NCH = DEPTH // _CD
_NBUF = 4
_CH = H // 4


def _mega_body(idx_smem, x_ref, w1_ref, b1_ref, w2_ref, b2_ref, wh_ref,
               bh_ref, idx2d_ref, stack_any, ns_any, logits_ref, value_ref,
               nidx_ref, top, h, p, buf, sem_g, sem_in, sem_out):
    def in_cp(c):
        b = c % _NBUF
        return pltpu.make_async_copy(
            stack_any.at[pl.ds(c * _CD, _CD)], buf.at[b], sem_in.at[b])

    def out_cp(c):
        b = c % _NBUF
        return pltpu.make_async_copy(
            buf.at[b], ns_any.at[pl.ds(c * _CD, _CD)], sem_out.at[b])

    def g_start(r, carry):
        d = idx_smem[r]
        pltpu.make_async_copy(stack_any.at[d, r], top.at[r], sem_g).start()
        return carry

    lax.fori_loop(0, B, g_start, 0)

    in_cp(0).start()
    in_cp(1).start()

    def g_drain(r, carry):
        d = idx_smem[r]
        pltpu.make_async_copy(stack_any.at[d, r], top.at[r], sem_g).wait()
        return carry

    def mlp_piece(c):
        if c == 3:
            lax.fori_loop(0, B, g_drain, 0)
        elif 4 <= c < 8:
            c0 = (c - 4) * _CH
            h[:, c0:c0 + _CH] = jnp.tanh(
                jnp.dot(x_ref[...], w1_ref[:OBS, c0:c0 + _CH],
                        preferred_element_type=jnp.float32)
                + jnp.dot(top[...], w1_ref[OBS:, c0:c0 + _CH],
                          preferred_element_type=jnp.float32)
                + b1_ref[:, c0:c0 + _CH]
            )
        elif 8 <= c < 12:
            c0 = (c - 8) * _CH
            p[:, c0:c0 + _CH] = jnp.tanh(
                jnp.dot(h[...], w2_ref[:, c0:c0 + _CH],
                        preferred_element_type=jnp.float32)
                + b2_ref[:, c0:c0 + _CH]
            )
        elif c == 12:
            ph = jnp.dot(p[...], wh_ref[...],
                         preferred_element_type=jnp.float32) + bh_ref[...]
            logits_ref[...] = ph[:, 3:3 + NOUT]
            value_ref[...] = ph[:, 3 + NOUT:4 + NOUT]
            s0 = ph[:, 0:1]
            s1 = ph[:, 1:2]
            s2 = ph[:, 2:3]
            op = jnp.where(s1 > s0, 1, 0)
            best = jnp.maximum(s0, s1)
            op = jnp.where(s2 > best, 2, op)
            nidx_ref[...] = jnp.maximum(idx2d_ref[...] + op - 1, 0)

    for c in range(_NCH):
        in_cp(c).wait()
        out_cp(c).start()
        if c + 2 < _NCH:
            if c >= 2:
                out_cp(c - 2).wait()
            in_cp(c + 2).start()
        mlp_piece(c)
    out_cp(_NCH - 4).wait()
    out_cp(_NCH - 3).wait()
    out_cp(_NCH - 2).wait()
    out_cp(_NCH - 1).wait()

    def s_start(r, carry):
        t = idx_smem[r] + 1
        pltpu.make_async_copy(p.at[r], ns_any.at[t, r], sem_g).start()
        return carry

    lax.fori_loop(0, B, s_start, 0)

    def s_drain(r, carry):
        t = idx_smem[r] + 1
        pltpu.make_async_copy(p.at[r], ns_any.at[t, r], sem_g).wait()
        return carry

    lax.fori_loop(0, B, s_drain, 0)


def _tc_mega(stack_t, stack_idx, x, w1, b1, w2, b2, wh, bh, idx2d):
    return pl.pallas_call(
        _mega_body,
        out_shape=(
            jax.ShapeDtypeStruct((DEPTH, B, H), jnp.float32),
            jax.ShapeDtypeStruct((B, NOUT), jnp.float32),
            jax.ShapeDtypeStruct((B, 1), jnp.float32),
            jax.ShapeDtypeStruct((B, 1), jnp.int32),
        ),
        in_specs=[
            pl.BlockSpec(memory_space=pltpu.SMEM),
            pl.BlockSpec(memory_space=pltpu.VMEM),
            pl.BlockSpec(memory_space=pltpu.VMEM),
            pl.BlockSpec(memory_space=pltpu.VMEM),
            pl.BlockSpec(memory_space=pltpu.VMEM),
            pl.BlockSpec(memory_space=pltpu.VMEM),
            pl.BlockSpec(memory_space=pltpu.VMEM),
            pl.BlockSpec(memory_space=pltpu.VMEM),
            pl.BlockSpec(memory_space=pltpu.VMEM),
            pl.BlockSpec(memory_space=pl.ANY),
        ],
        out_specs=[
            pl.BlockSpec(memory_space=pl.ANY),
            pl.BlockSpec(memory_space=pltpu.VMEM),
            pl.BlockSpec(memory_space=pltpu.VMEM),
            pl.BlockSpec(memory_space=pltpu.VMEM),
        ],
        scratch_shapes=[
            pltpu.VMEM((B, H), jnp.float32),
            pltpu.VMEM((B, H), jnp.float32),
            pltpu.VMEM((B, H), jnp.float32),
            pltpu.VMEM((_NBUF, _CD, B, H), jnp.float32),
            pltpu.SemaphoreType.DMA,
            pltpu.SemaphoreType.DMA((_NBUF,)),
            pltpu.SemaphoreType.DMA((_NBUF,)),
        ],
    )(stack_idx, x, w1, b1, w2, b2, wh, bh, idx2d, stack_t)


def kernel(x, stack, stack_idx, W1, b1, W2, b2, Ws, bs, Wp, bp, Wv, bv):
    idx2d = stack_idx.reshape(B, 1)
    stack_t = jnp.swapaxes(stack, 0, 1)

    wh = jnp.concatenate(
        [Ws, Wp, Wv, jnp.zeros((H, HEADW - NOUT - 4), jnp.float32)], axis=1)
    bh = jnp.concatenate(
        [bs, bp, bv, jnp.zeros((HEADW - NOUT - 4,), jnp.float32)]
    ).reshape(1, HEADW)

    new_stack_t, logits, value, nidx = _tc_mega(
        stack_t, stack_idx, x, W1, b1.reshape(1, H), W2, b2.reshape(1, H),
        wh, bh, idx2d,
    )
    new_stack = jnp.swapaxes(new_stack_t, 0, 1)

    return (logits, value[:, 0], new_stack, nidx[:, 0])


# speedup vs baseline: 1.0882x; 1.0106x over previous
import jax
import jax.numpy as jnp
from jax import lax
from jax.experimental import pallas as pl
from jax.experimental.pallas import tpu as pltpu

B = 256
OBS = 1024
H = 2048
DEPTH = 100
NOUT = 64
HEADW = 128

_CD = 1
_NCH = DEPTH // _CD
_NBUF = 4
_CH = H // 4


def _mega_body(idx_smem, x_ref, w1_any, b1_ref, w2_any, b2_ref, wh_any,
               bh_ref, idx2d_ref, stack_any, ns_any, logits_ref, value_ref,
               nidx_ref, top, h, p, buf, w1_ref, w2_ref, wh_ref, sem_g,
               sem_w, sem_in, sem_out):
    def in_cp(c):
        b = c % _NBUF
        return pltpu.make_async_copy(
            stack_any.at[pl.ds(c * _CD, _CD)], buf.at[b], sem_in.at[b])

    def out_cp(c):
        b = c % _NBUF
        return pltpu.make_async_copy(
            buf.at[b], ns_any.at[pl.ds(c * _CD, _CD)], sem_out.at[b])

    in_cp(0).start()
    in_cp(1).start()

    w1_cp = pltpu.make_async_copy(w1_any, w1_ref, sem_w.at[0])
    w2_cp = pltpu.make_async_copy(w2_any, w2_ref, sem_w.at[1])
    wh_cp = pltpu.make_async_copy(wh_any, wh_ref, sem_w.at[2])
    w1_cp.start()
    w2_cp.start()
    wh_cp.start()

    def g_start(r, carry):
        d = idx_smem[r]
        pltpu.make_async_copy(stack_any.at[d, r], top.at[r], sem_g).start()
        return carry

    lax.fori_loop(0, B, g_start, 0)

    def g_drain(r, carry):
        d = idx_smem[r]
        pltpu.make_async_copy(stack_any.at[d, r], top.at[r], sem_g).wait()
        return carry

    def mlp_piece(c):
        if c == 8:
            lax.fori_loop(0, B, g_drain, 0)
            w1_cp.wait()
        elif 9 <= c < 13:
            c0 = (c - 9) * _CH
            h[:, c0:c0 + _CH] = jnp.tanh(
                jnp.dot(x_ref[...], w1_ref[:OBS, c0:c0 + _CH],
                        preferred_element_type=jnp.float32)
                + jnp.dot(top[...], w1_ref[OBS:, c0:c0 + _CH],
                          preferred_element_type=jnp.float32)
                + b1_ref[:, c0:c0 + _CH]
            )
        elif c == 13:
            w2_cp.wait()
        elif 14 <= c < 18:
            c0 = (c - 14) * _CH
            p[:, c0:c0 + _CH] = jnp.tanh(
                jnp.dot(h[...], w2_ref[:, c0:c0 + _CH],
                        preferred_element_type=jnp.float32)
                + b2_ref[:, c0:c0 + _CH]
            )
        elif c == 18:
            wh_cp.wait()
            pass
        elif c == 19:
            ph = jnp.dot(p[...], wh_ref[...],
                         preferred_element_type=jnp.float32) + bh_ref[...]
            logits_ref[...] = ph[:, 3:3 + NOUT]
            value_ref[...] = ph[:, 3 + NOUT:4 + NOUT]
            s0 = ph[:, 0:1]
            s1 = ph[:, 1:2]
            s2 = ph[:, 2:3]
            op = jnp.where(s1 > s0, 1, 0)
            best = jnp.maximum(s0, s1)
            op = jnp.where(s2 > best, 2, op)
            nidx_ref[...] = jnp.maximum(idx2d_ref[...] + op - 1, 0)

    for c in range(_NCH):
        in_cp(c).wait()
        out_cp(c).start()
        if c + 2 < _NCH:
            if c >= 2:
                out_cp(c - 2).wait()
            in_cp(c + 2).start()
        mlp_piece(c)
    out_cp(_NCH - 4).wait()
    out_cp(_NCH - 3).wait()
    out_cp(_NCH - 2).wait()
    out_cp(_NCH - 1).wait()

    def s_start(r, carry):
        t = idx_smem[r] + 1
        pltpu.make_async_copy(p.at[r], ns_any.at[t, r], sem_g).start()
        return carry

    lax.fori_loop(0, B, s_start, 0)

    def s_drain(r, carry):
        t = idx_smem[r] + 1
        pltpu.make_async_copy(p.at[r], ns_any.at[t, r], sem_g).wait()
        return carry

    lax.fori_loop(0, B, s_drain, 0)


def _tc_mega(stack_t, stack_idx, x, w1, b1, w2, b2, wh, bh, idx2d):
    return pl.pallas_call(
        _mega_body,
        out_shape=(
            jax.ShapeDtypeStruct((DEPTH, B, H), jnp.float32),
            jax.ShapeDtypeStruct((B, NOUT), jnp.float32),
            jax.ShapeDtypeStruct((B, 1), jnp.float32),
            jax.ShapeDtypeStruct((B, 1), jnp.int32),
        ),
        in_specs=[
            pl.BlockSpec(memory_space=pltpu.SMEM),
            pl.BlockSpec(memory_space=pltpu.VMEM),
            pl.BlockSpec(memory_space=pl.ANY),
            pl.BlockSpec(memory_space=pltpu.VMEM),
            pl.BlockSpec(memory_space=pl.ANY),
            pl.BlockSpec(memory_space=pltpu.VMEM),
            pl.BlockSpec(memory_space=pl.ANY),
            pl.BlockSpec(memory_space=pltpu.VMEM),
            pl.BlockSpec(memory_space=pltpu.VMEM),
            pl.BlockSpec(memory_space=pl.ANY),
        ],
        out_specs=[
            pl.BlockSpec(memory_space=pl.ANY),
            pl.BlockSpec(memory_space=pltpu.VMEM),
            pl.BlockSpec(memory_space=pltpu.VMEM),
            pl.BlockSpec(memory_space=pltpu.VMEM),
        ],
        scratch_shapes=[
            pltpu.VMEM((B, H), jnp.float32),
            pltpu.VMEM((B, H), jnp.float32),
            pltpu.VMEM((B, H), jnp.float32),
            pltpu.VMEM((_NBUF, _CD, B, H), jnp.float32),
            pltpu.VMEM((OBS + H, H), jnp.float32),
            pltpu.VMEM((H, H), jnp.float32),
            pltpu.VMEM((H, HEADW), jnp.float32),
            pltpu.SemaphoreType.DMA,
            pltpu.SemaphoreType.DMA((3,)),
            pltpu.SemaphoreType.DMA((_NBUF,)),
            pltpu.SemaphoreType.DMA((_NBUF,)),
        ],
    )(stack_idx, x, w1, b1, w2, b2, wh, bh, idx2d, stack_t)


def kernel(x, stack, stack_idx, W1, b1, W2, b2, Ws, bs, Wp, bp, Wv, bv):
    idx2d = stack_idx.reshape(B, 1)
    stack_t = jnp.swapaxes(stack, 0, 1)

    wh = jnp.concatenate(
        [Ws, Wp, Wv, jnp.zeros((H, HEADW - NOUT - 4), jnp.float32)], axis=1)
    bh = jnp.concatenate(
        [bs, bp, bv, jnp.zeros((HEADW - NOUT - 4,), jnp.float32)]
    ).reshape(1, HEADW)

    new_stack_t, logits, value, nidx = _tc_mega(
        stack_t, stack_idx, x, W1, b1.reshape(1, H), W2, b2.reshape(1, H),
        wh, bh, idx2d,
    )
    new_stack = jnp.swapaxes(new_stack_t, 0, 1)

    return (logits, value[:, 0], new_stack, nidx[:, 0])
